# Initial kernel scaffold; baseline (speedup 1.0000x reference)
import jax
import jax.numpy as jnp
from jax.experimental import pallas as pl


def kernel(x_numeric, x_categorical, lin_tables, int_tables, W_num, b_num, num_vectors, bias):
    raise NotImplementedError("write your pallas kernel here")


# speedup vs baseline: 1.4408x; 1.4408x over previous
import functools

import jax
import jax.numpy as jnp
from jax import lax
from jax.experimental import pallas as pl
from jax.experimental.pallas import tpu as pltpu
from jax.experimental.pallas import tpu_sc as plsc

B = 16384
F = 26
V = 100000
D = 16
N_NUM = 13

NC = 2
NS = 16
NW = NC * NS
RPW = B // NW
IPW = RPW * F
IDX_ROWS = IPW // 128
G = 64
IG = G * F
GR = IG // 128
NG = RPW // G


def _sc_body(idx_hbm, int_hbm, lin_hbm, s_out, q_out, lin_out,
             idx_v, rows_v, lin_v, s_buf, q_buf, sem):
    wid = lax.axis_index("s") * NC + lax.axis_index("c")
    base_row = wid * RPW

    pltpu.sync_copy(idx_hbm.at[wid], idx_v)

    def group_body(g, _):
        copies = []
        for j in range(GR):
            idx_row = idx_v.at[g * GR + j]
            copies.append(pltpu.async_copy(
                int_hbm.at[idx_row], rows_v.at[pl.ds(j * 128, 128)], sem))
            copies.append(pltpu.async_copy(
                lin_hbm.at[idx_row], lin_v.at[pl.ds(j * 128, 128)], sem))
        for c in copies:
            c.wait()

        pltpu.sync_copy(lin_v, lin_out.at[pl.ds(wid * IPW + g * IG, IG)])

        def row_body(r, _):
            v = rows_v[r * F, :]
            s = v
            q = v * v
            for k in range(1, F):
                v = rows_v[r * F + k, :]
                s = s + v
                q = q + v * v
            s_buf[g * G + r, :] = s
            q_buf[g * G + r, :] = q
            return _

        lax.fori_loop(0, G, row_body, None)
        return _

    lax.fori_loop(0, NG, group_body, None)

    pltpu.sync_copy(s_buf, s_out.at[pl.ds(base_row, RPW)])
    pltpu.sync_copy(q_buf, q_out.at[pl.ds(base_row, RPW)])


@functools.lru_cache(maxsize=1)
def _make_sc_gather():
    return pl.kernel(
        _sc_body,
        out_type=[
            jax.ShapeDtypeStruct((B, D), jnp.float32),
            jax.ShapeDtypeStruct((B, D), jnp.float32),
            jax.ShapeDtypeStruct((B * F,), jnp.float32),
        ],
        mesh=plsc.VectorSubcoreMesh(
            core_axis_name="c", subcore_axis_name="s",
            num_cores=NC, num_subcores=NS),
        scratch_types=[
            pltpu.VMEM((IDX_ROWS, 128), jnp.int32),
            pltpu.VMEM((IG, D), jnp.float32),
            pltpu.VMEM((IG,), jnp.float32),
            pltpu.VMEM((RPW, D), jnp.float32),
            pltpu.VMEM((RPW, D), jnp.float32),
            pltpu.SemaphoreType.DMA,
        ],
        compiler_params=pltpu.CompilerParams(use_tc_tiling_on_sc=False),
    )


BT = 2048


def _tc_body(x_ref, lin_ref, s_ref, q_ref, nv_ref, w_ref, c0_ref, o_ref):
    x = x_ref[:]
    nv = nv_ref[:]
    n = jnp.zeros((BT, D), jnp.float32)
    sqn = jnp.zeros((BT, D), jnp.float32)
    linw = jnp.zeros((BT, 1), jnp.float32)
    for j in range(N_NUM):
        xj = x[:, j:j + 1]
        nvj = nv[j:j + 1, :]
        n = n + xj * nvj
        sqn = sqn + (xj * xj) * (nvj * nvj)
        linw = linw + xj * w_ref[0, j]
    stot = s_ref[:] + n
    inter = 0.5 * (
        jnp.sum(stot * stot, axis=1, keepdims=True)
        - jnp.sum(q_ref[:], axis=1, keepdims=True)
        - jnp.sum(sqn, axis=1, keepdims=True))
    lin = jnp.sum(lin_ref[:], axis=1, keepdims=True) + linw + c0_ref[0, 0]
    o_ref[:] = lin + inter


def _tc_combine(x_numeric, lin_raw, s, q, num_vectors, W_num, c0):
    grid = (B // BT,)
    return pl.pallas_call(
        _tc_body,
        grid=grid,
        in_specs=[
            pl.BlockSpec((BT, N_NUM), lambda i: (i, 0)),
            pl.BlockSpec((BT, F), lambda i: (i, 0)),
            pl.BlockSpec((BT, D), lambda i: (i, 0)),
            pl.BlockSpec((BT, D), lambda i: (i, 0)),
            pl.BlockSpec((N_NUM, D), lambda i: (0, 0)),
            pl.BlockSpec((1, N_NUM), lambda i: (0, 0), memory_space=pltpu.SMEM),
            pl.BlockSpec((1, 1), lambda i: (0, 0), memory_space=pltpu.SMEM),
        ],
        out_specs=pl.BlockSpec((BT, 1), lambda i: (i, 0)),
        out_shape=jax.ShapeDtypeStruct((B, 1), jnp.float32),
    )(x_numeric, lin_raw, s, q, num_vectors, W_num, c0)


@jax.jit
def kernel(x_numeric, x_categorical, lin_tables, int_tables, W_num, b_num,
           num_vectors, bias):
    offs = (jnp.arange(F, dtype=jnp.int32) * V)[None, :]
    flat_idx = x_categorical + offs
    idx3 = flat_idx.reshape(NW, IDX_ROWS, 128)
    int_flat = int_tables.reshape(F * V, D)
    lin_flat = lin_tables.reshape(F * V)

    s, q, lin_raw = _make_sc_gather()(idx3, int_flat, lin_flat)

    c0 = (bias + b_num).reshape(1, 1)
    out = _tc_combine(x_numeric, lin_raw.reshape(B, F), s, q,
                      num_vectors, W_num, c0)
    return out[:, 0]


# speedup vs baseline: 1.9297x; 1.3393x over previous
import functools

import jax
import jax.numpy as jnp
from jax import lax
from jax.experimental import pallas as pl
from jax.experimental.pallas import tpu as pltpu
from jax.experimental.pallas import tpu_sc as plsc

B = 16384
F = 26
V = 100000
D = 16
N_NUM = 13

NC = 2
NS = 16
NW = NC * NS
RPW = B // NW
G = 128
NG = RPW // G
IG = G * F
LIN_ROWS = RPW * F // 128


def _sc_body(tab2, lin1, idxv_h, idxlin_h, s_out, q_out, lin_out,
             idxv_v, idxlin_v, pg, lin_v, s_buf, q_buf, sem):
    wid = lax.axis_index("s") * NC + lax.axis_index("c")
    base_row = wid * RPW

    pltpu.sync_copy(idxv_h.at[wid], idxv_v)
    pltpu.sync_copy(idxlin_h.at[wid], idxlin_v)

    iota_d = lax.iota(jnp.int32, 16) * G

    def group_body(g, _):
        def fire_fd(t, _c):
            f = t // D
            pltpu.async_copy(tab2.at[t].at[idxv_v.at[g, f]],
                             pg.at[pl.ds(t * G, G)], sem)
            return _c

        lax.fori_loop(0, F * D, fire_fd, None)

        def fire_lin(j, _c):
            pltpu.async_copy(lin1.at[idxlin_v.at[g * F + j]],
                             lin_v.at[pl.ds(j * 128, 128)], sem)
            return _c

        lax.fori_loop(0, F, fire_lin, None)

        def drain(j, _c):
            pltpu.make_async_copy(lin1.at[pl.ds(0, 128)],
                                  lin_v.at[pl.ds(0, 128)], sem).wait()
            return _c

        lax.fori_loop(0, F * D + F, drain, None)

        pltpu.sync_copy(lin_v, lin_out.at[pl.ds(wid * RPW * F + g * IG, IG)])

        def row_body(r, _c):
            row_vec = iota_d + r
            v = plsc.load_gather(pg, [row_vec])
            s = v
            q = v * v
            for f in range(1, F):
                v = plsc.load_gather(pg, [row_vec + (f * D * G)])
                s = s + v
                q = q + v * v
            s_buf[g * G + r, :] = s
            q_buf[g * G + r, :] = q
            return _c

        lax.fori_loop(0, G, row_body, None)
        return _

    lax.fori_loop(0, NG, group_body, None)

    pltpu.sync_copy(s_buf, s_out.at[pl.ds(base_row, RPW)])
    pltpu.sync_copy(q_buf, q_out.at[pl.ds(base_row, RPW)])


@functools.lru_cache(maxsize=1)
def _make_sc_gather():
    return pl.kernel(
        _sc_body,
        out_type=[
            jax.ShapeDtypeStruct((B, D), jnp.float32),
            jax.ShapeDtypeStruct((B, D), jnp.float32),
            jax.ShapeDtypeStruct((B * F,), jnp.float32),
        ],
        mesh=plsc.VectorSubcoreMesh(
            core_axis_name="c", subcore_axis_name="s",
            num_cores=NC, num_subcores=NS),
        scratch_types=[
            pltpu.VMEM((NG, F, 128), jnp.int32),
            pltpu.VMEM((LIN_ROWS, 128), jnp.int32),
            pltpu.VMEM((F * D * G,), jnp.float32),
            pltpu.VMEM((IG,), jnp.float32),
            pltpu.VMEM((RPW, D), jnp.float32),
            pltpu.VMEM((RPW, D), jnp.float32),
            pltpu.SemaphoreType.DMA,
        ],
        compiler_params=pltpu.CompilerParams(
            use_tc_tiling_on_sc=False, needs_layout_passes=False),
    )


BT = 2048


def _tc_body(x_ref, lin_ref, s_ref, q_ref, nv_ref, w_ref, c0_ref, o_ref):
    x = x_ref[:]
    nv = nv_ref[:]
    n = jnp.zeros((BT, D), jnp.float32)
    sqn = jnp.zeros((BT, D), jnp.float32)
    linw = jnp.zeros((BT, 1), jnp.float32)
    for j in range(N_NUM):
        xj = x[:, j:j + 1]
        nvj = nv[j:j + 1, :]
        n = n + xj * nvj
        sqn = sqn + (xj * xj) * (nvj * nvj)
        linw = linw + xj * w_ref[0, j]
    stot = s_ref[:] + n
    inter = 0.5 * (
        jnp.sum(stot * stot, axis=1, keepdims=True)
        - jnp.sum(q_ref[:], axis=1, keepdims=True)
        - jnp.sum(sqn, axis=1, keepdims=True))
    lin = jnp.sum(lin_ref[:], axis=1, keepdims=True) + linw + c0_ref[0, 0]
    o_ref[:] = lin + inter


def _tc_combine(x_numeric, lin_raw, s, q, num_vectors, W_num, c0):
    grid = (B // BT,)
    return pl.pallas_call(
        _tc_body,
        grid=grid,
        in_specs=[
            pl.BlockSpec((BT, N_NUM), lambda i: (i, 0)),
            pl.BlockSpec((BT, F), lambda i: (i, 0)),
            pl.BlockSpec((BT, D), lambda i: (i, 0)),
            pl.BlockSpec((BT, D), lambda i: (i, 0)),
            pl.BlockSpec((N_NUM, D), lambda i: (0, 0)),
            pl.BlockSpec((1, N_NUM), lambda i: (0, 0), memory_space=pltpu.SMEM),
            pl.BlockSpec((1, 1), lambda i: (0, 0), memory_space=pltpu.SMEM),
        ],
        out_specs=pl.BlockSpec((BT, 1), lambda i: (i, 0)),
        out_shape=jax.ShapeDtypeStruct((B, 1), jnp.float32),
    )(x_numeric, lin_raw, s, q, num_vectors, W_num, c0)


@jax.jit
def kernel(x_numeric, x_categorical, lin_tables, int_tables, W_num, b_num,
           num_vectors, bias):
    tab2 = jnp.swapaxes(int_tables, 1, 2).reshape(F * D, V)
    lin1 = lin_tables.reshape(F * V)
    idxv = x_categorical.reshape(NW, NG, G, F).transpose(0, 1, 3, 2)
    offs = (jnp.arange(F, dtype=jnp.int32) * V)[None, :]
    idxlin = (x_categorical + offs).reshape(NW, LIN_ROWS, 128)

    s, q, lin_raw = _make_sc_gather()(tab2, lin1, idxv, idxlin)

    c0 = (bias + b_num).reshape(1, 1)
    out = _tc_combine(x_numeric, lin_raw.reshape(B, F), s, q,
                      num_vectors, W_num, c0)
    return out[:, 0]


# speedup vs baseline: 2.1562x; 1.1174x over previous
import functools

import jax
import jax.numpy as jnp
from jax import lax
from jax.experimental import pallas as pl
from jax.experimental.pallas import tpu as pltpu
from jax.experimental.pallas import tpu_sc as plsc

B = 16384
F = 26
V = 100000
D = 16
N_NUM = 13

NC = 2
NS = 16
NW = NC * NS
RPW = B // NW
G = 128
NG = RPW // G
IG = G * F
LIN_ROWS = RPW * F // 128


def _sc_body(tab2, lin1, idxlin_h, s_out, q_out, lin_out,
             idxlin_v, ivx, pg, lin_v, s_buf, q_buf, sem):
    wid = lax.axis_index("s") * NC + lax.axis_index("c")
    base_row = wid * RPW

    pltpu.sync_copy(idxlin_h.at[wid], idxlin_v)

    iota_d = lax.iota(jnp.int32, 16) * G
    iota_f = lax.iota(jnp.int32, 16) * F

    def group_body(g, _):
        for f in range(F):
            for k in range(G // 16):
                p0 = (g * G + k * 16) * F + f
                pvec = iota_f + p0
                ivx[f, pl.ds(k * 16, 16)] = (
                    plsc.load_gather(idxlin_v, [pvec]) - f * V)

        def fire_fd(t, _c):
            f = t // D
            pltpu.async_copy(tab2.at[t].at[ivx.at[f]],
                             pg.at[pl.ds(t * G, G)], sem)
            return _c

        lax.fori_loop(0, F * D, fire_fd, None)

        def fire_lin(j, _c):
            pltpu.async_copy(lin1.at[idxlin_v.at[pl.ds((g * F + j) * 128, 128)]],
                             lin_v.at[pl.ds(j * 128, 128)], sem)
            return _c

        lax.fori_loop(0, F, fire_lin, None)

        def drain(j, _c):
            pltpu.make_async_copy(lin1.at[pl.ds(0, 128)],
                                  lin_v.at[pl.ds(0, 128)], sem).wait()
            return _c

        lax.fori_loop(0, F * D + F, drain, None)

        pltpu.sync_copy(lin_v, lin_out.at[pl.ds(wid * RPW * F + g * IG, IG)])

        def row_body(r, _c):
            row_vec = iota_d + r
            v = plsc.load_gather(pg, [row_vec])
            s = v
            q = v * v
            for f in range(1, F):
                v = plsc.load_gather(pg, [row_vec + (f * D * G)])
                s = s + v
                q = q + v * v
            s_buf[g * G + r, :] = s
            q_buf[g * G + r, :] = q
            return _c

        lax.fori_loop(0, G, row_body, None)
        return _

    lax.fori_loop(0, NG, group_body, None)

    pltpu.sync_copy(s_buf, s_out.at[pl.ds(base_row, RPW)])
    pltpu.sync_copy(q_buf, q_out.at[pl.ds(base_row, RPW)])


@functools.lru_cache(maxsize=1)
def _make_sc_gather():
    return pl.kernel(
        _sc_body,
        out_type=[
            jax.ShapeDtypeStruct((B, D), jnp.float32),
            jax.ShapeDtypeStruct((B, D), jnp.float32),
            jax.ShapeDtypeStruct((B * F,), jnp.float32),
        ],
        mesh=plsc.VectorSubcoreMesh(
            core_axis_name="c", subcore_axis_name="s",
            num_cores=NC, num_subcores=NS),
        scratch_types=[
            pltpu.VMEM((RPW * F,), jnp.int32),
            pltpu.VMEM((F, 128), jnp.int32),
            pltpu.VMEM((F * D * G,), jnp.float32),
            pltpu.VMEM((IG,), jnp.float32),
            pltpu.VMEM((RPW, D), jnp.float32),
            pltpu.VMEM((RPW, D), jnp.float32),
            pltpu.SemaphoreType.DMA,
        ],
        compiler_params=pltpu.CompilerParams(
            use_tc_tiling_on_sc=False, needs_layout_passes=False),
    )


BT = 2048


def _tc_body(x_ref, lin_ref, s_ref, q_ref, nv_ref, w_ref, c0_ref, o_ref):
    x = x_ref[:]
    nv = nv_ref[:]
    dn = (((1,), (0,)), ((), ()))
    hi = lax.Precision.HIGHEST
    n = lax.dot_general(x, nv, dn, precision=hi,
                        preferred_element_type=jnp.float32)
    sqn = lax.dot_general(x * x, nv * nv, dn, precision=hi,
                          preferred_element_type=jnp.float32)
    linw = lax.dot_general(x, w_ref[:], (((1,), (1,)), ((), ())),
                           precision=hi,
                           preferred_element_type=jnp.float32)
    stot = s_ref[:] + n
    t = stot * stot - q_ref[:] - sqn
    inter = 0.5 * jnp.sum(t, axis=1, keepdims=True)
    lin = jnp.sum(lin_ref[:], axis=1, keepdims=True) + linw + c0_ref[0, 0]
    o_ref[:] = lin + inter


def _tc_combine(x_numeric, lin_raw, s, q, num_vectors, W_num, c0):
    grid = (B // BT,)
    return pl.pallas_call(
        _tc_body,
        grid=grid,
        in_specs=[
            pl.BlockSpec((BT, N_NUM), lambda i: (i, 0)),
            pl.BlockSpec((BT, F), lambda i: (i, 0)),
            pl.BlockSpec((BT, D), lambda i: (i, 0)),
            pl.BlockSpec((BT, D), lambda i: (i, 0)),
            pl.BlockSpec((N_NUM, D), lambda i: (0, 0)),
            pl.BlockSpec((1, N_NUM), lambda i: (0, 0)),
            pl.BlockSpec((1, 1), lambda i: (0, 0), memory_space=pltpu.SMEM),
        ],
        out_specs=pl.BlockSpec((BT, 1), lambda i: (i, 0)),
        out_shape=jax.ShapeDtypeStruct((B, 1), jnp.float32),
    )(x_numeric, lin_raw, s, q, num_vectors, W_num, c0)


@jax.jit
def kernel(x_numeric, x_categorical, lin_tables, int_tables, W_num, b_num,
           num_vectors, bias):
    tab2 = jnp.swapaxes(int_tables, 1, 2).reshape(F * D, V)
    lin1 = lin_tables.reshape(F * V)
    offs = (jnp.arange(F, dtype=jnp.int32) * V)[None, :]
    idxlin = (x_categorical + offs).reshape(NW, RPW * F)

    s, q, lin_raw = _make_sc_gather()(tab2, lin1, idxlin)

    c0 = (bias + b_num).reshape(1, 1)
    out = _tc_combine(x_numeric, lin_raw.reshape(B, F), s, q,
                      num_vectors, W_num, c0)
    return out[:, 0]
